# baseline (speedup 1.0000x reference)
import functools

import jax
import jax.numpy as jnp
from jax.experimental import pallas as pl
from jax.experimental.pallas import tpu as pltpu
from jax.experimental.pallas import tpu_sc as plsc

N = 10000
E = 320000
EH = E // 2
H = 128
DE = 16
AVG_DEG = float(E) / float(N)

NC = 2
NS = 16
NW = NC * NS

GCH = 64
GROWS_H = EH // GCH
GRPT = 80
GRPT_LAST = GROWS_H - (NW - 1) * GRPT

SCH = 128
SROWS_H = EH // SCH
SRPT = 40
SRPT_LAST = SROWS_H - (NW - 1) * SRPT

BN = 1000
BE = 1280
NPT = 624
NREM = N - NS * NPT

_f32 = jnp.float32


def _silu(v):
    return v * jax.nn.sigmoid(v)


def _mesh():
    return plsc.VectorSubcoreMesh(core_axis_name="c", subcore_axis_name="s",
                                  num_cores=NC, num_subcores=NS)


def _prep_body(h_ref, win_ref, bin_ref, a_ref, b_ref,
               h1_ref, tp_ref, tq_ref):
    h1 = jnp.dot(h_ref[...], win_ref[...],
                 preferred_element_type=_f32) + bin_ref[...]
    h1_ref[...] = h1
    tp_ref[...] = jnp.dot(h1, a_ref[...], preferred_element_type=_f32)
    tq_ref[...] = jnp.dot(h1, b_ref[...], preferred_element_type=_f32)


def _tc_prep(h, w_in, b_in, a0, b0):
    grid = (N // BN,)
    blk = lambda i: (i, 0)
    full = lambda i: (0, 0)
    return pl.pallas_call(
        _prep_body,
        grid=grid,
        in_specs=[
            pl.BlockSpec((BN, H), blk),
            pl.BlockSpec((H, H), full),
            pl.BlockSpec((1, H), full),
            pl.BlockSpec((H, H), full),
            pl.BlockSpec((H, H), full),
        ],
        out_specs=[
            pl.BlockSpec((BN, H), blk),
            pl.BlockSpec((BN, H), blk),
            pl.BlockSpec((BN, H), blk),
        ],
        out_shape=[
            jax.ShapeDtypeStruct((N, H), _f32),
            jax.ShapeDtypeStruct((N, H), _f32),
            jax.ShapeDtypeStruct((N, H), _f32),
        ],
        compiler_params=pltpu.CompilerParams(
            dimension_semantics=("arbitrary",)),
    )(h, w_in, b_in, a0, b0)


def _edge0_body(ps_ref, xs_ref, qd_ref, xd_ref, ea_ref, wr_ref, wea_ref,
                be1_ref, we2_ref, be2_ref, wx1_ref, bx1_ref, wx2t_ref,
                bx2_ref, ones_ref, m_ref, t_ref):
    ps = ps_ref[...]
    qd = qd_ref[...]
    diff = xs_ref[...] - xd_ref[...]
    r2b = jnp.dot(diff * diff, ones_ref[...], preferred_element_type=_f32,
                  precision=jax.lax.Precision.HIGHEST)
    a = (ps + qd + r2b * wr_ref[...] +
         jnp.dot(ea_ref[...], wea_ref[...], preferred_element_type=_f32) +
         be1_ref[...])
    m = _silu(jnp.dot(_silu(a), we2_ref[...],
                      preferred_element_type=_f32) + be2_ref[...])
    u = _silu(jnp.dot(m, wx1_ref[...],
                      preferred_element_type=_f32) + bx1_ref[...])
    wb = jnp.dot(u, wx2t_ref[...], preferred_element_type=_f32) + bx2_ref[...]
    m_ref[...] = m
    t_ref[...] = diff * wb


def _tc_edge0(gps, gxs, gqd, gxd, ea, wr, wea, be1, we2, be2, wx1, bx1,
              wx2t, bx2, ones_c):
    grid = (EH // BE,)
    blk = lambda i: (i, 0)
    full = lambda i: (0, 0)
    return pl.pallas_call(
        _edge0_body,
        grid=grid,
        in_specs=[
            pl.BlockSpec((BE, H), blk),
            pl.BlockSpec((BE, H), blk),
            pl.BlockSpec((BE, H), blk),
            pl.BlockSpec((BE, H), blk),
            pl.BlockSpec((BE, DE), blk),
            pl.BlockSpec((1, H), full),
            pl.BlockSpec((DE, H), full),
            pl.BlockSpec((1, H), full),
            pl.BlockSpec((H, H), full),
            pl.BlockSpec((1, H), full),
            pl.BlockSpec((H, H), full),
            pl.BlockSpec((1, H), full),
            pl.BlockSpec((H, H), full),
            pl.BlockSpec((1, 1), full),
            pl.BlockSpec((H, H), full),
        ],
        out_specs=[
            pl.BlockSpec((BE, H), blk),
            pl.BlockSpec((BE, H), blk),
        ],
        out_shape=[
            jax.ShapeDtypeStruct((EH, H), _f32),
            jax.ShapeDtypeStruct((EH, H), _f32),
        ],
        compiler_params=pltpu.CompilerParams(
            dimension_semantics=("arbitrary",)),
    )(gps, gxs, gqd, gxd, ea, wr, wea, be1, we2, be2, wx1, bx1, wx2t, bx2,
      ones_c)


def _edge1_body(ps_ref, xs_ref, qd_ref, xd_ref, ea_ref, wr_ref, wea_ref,
                be1_ref, we2_ref, be2_ref, ones_ref, m_ref):
    diff = xs_ref[...] - xd_ref[...]
    r2b = jnp.dot(diff * diff, ones_ref[...], preferred_element_type=_f32,
                  precision=jax.lax.Precision.HIGHEST)
    a = (ps_ref[...] + qd_ref[...] + r2b * wr_ref[...] +
         jnp.dot(ea_ref[...], wea_ref[...], preferred_element_type=_f32) +
         be1_ref[...])
    m_ref[...] = _silu(jnp.dot(_silu(a), we2_ref[...],
                               preferred_element_type=_f32) + be2_ref[...])


def _tc_edge1(gps, gxs, gqd, gxd, ea, wr, wea, be1, we2, be2, ones_c):
    grid = (EH // BE,)
    blk = lambda i: (i, 0)
    full = lambda i: (0, 0)
    return pl.pallas_call(
        _edge1_body,
        grid=grid,
        in_specs=[
            pl.BlockSpec((BE, H), blk),
            pl.BlockSpec((BE, H), blk),
            pl.BlockSpec((BE, H), blk),
            pl.BlockSpec((BE, H), blk),
            pl.BlockSpec((BE, DE), blk),
            pl.BlockSpec((1, H), full),
            pl.BlockSpec((DE, H), full),
            pl.BlockSpec((1, H), full),
            pl.BlockSpec((H, H), full),
            pl.BlockSpec((1, H), full),
            pl.BlockSpec((H, H), full),
        ],
        out_specs=[pl.BlockSpec((BE, H), blk)],
        out_shape=[jax.ShapeDtypeStruct((EH, H), _f32)],
        compiler_params=pltpu.CompilerParams(
            dimension_semantics=("arbitrary",)),
    )(gps, gxs, gqd, gxd, ea, wr, wea, be1, we2, be2, ones_c)[0]


def _node0_body(h1_ref, xp_ref, aggha_ref, agghb_ref, aggxa_ref, aggxb_ref,
                wh1a_ref, wh1b_ref, bh1_ref, wh2_ref, bh2_ref, a1_ref, b1_ref,
                h2_ref, tp_ref, tq_ref, tx_ref):
    h1 = h1_ref[...]
    aggh = ((aggha_ref[0] + aggha_ref[1]) + (agghb_ref[0] + agghb_ref[1]))
    pre = (jnp.dot(h1, wh1a_ref[...], preferred_element_type=_f32) +
           jnp.dot(aggh, wh1b_ref[...], preferred_element_type=_f32) +
           bh1_ref[...])
    upd = jnp.dot(_silu(pre), wh2_ref[...],
                  preferred_element_type=_f32) + bh2_ref[...]
    h2 = h1 + upd
    h2_ref[...] = h2
    aggx = ((aggxa_ref[0] + aggxa_ref[1]) + (aggxb_ref[0] + aggxb_ref[1]))
    xn = xp_ref[...] + aggx * (1.0 / AVG_DEG)
    tp_ref[...] = jnp.dot(h2, a1_ref[...], preferred_element_type=_f32)
    tq_ref[...] = jnp.dot(h2, b1_ref[...], preferred_element_type=_f32)
    tx_ref[...] = xn


def _tc_node0(h1, xpad, aggha, agghb, aggxa, aggxb,
              wh1a, wh1b, bh1, wh2, bh2, a1, b1):
    grid = (N // BN,)
    blk = lambda i: (i, 0)
    blk3 = lambda i: (i, 0, 0)
    blk3c = lambda i: (0, i, 0)
    full = lambda i: (0, 0)
    return pl.pallas_call(
        _node0_body,
        grid=grid,
        in_specs=[
            pl.BlockSpec((BN, H), blk),
            pl.BlockSpec((BN, H), blk),
            pl.BlockSpec((NC, BN, H), blk3c),
            pl.BlockSpec((NC, BN, H), blk3c),
            pl.BlockSpec((NC, BN, H), blk3c),
            pl.BlockSpec((NC, BN, H), blk3c),
            pl.BlockSpec((H, H), full),
            pl.BlockSpec((H, H), full),
            pl.BlockSpec((1, H), full),
            pl.BlockSpec((H, H), full),
            pl.BlockSpec((1, H), full),
            pl.BlockSpec((H, H), full),
            pl.BlockSpec((H, H), full),
        ],
        out_specs=[
            pl.BlockSpec((BN, H), blk),
            pl.BlockSpec((BN, H), blk),
            pl.BlockSpec((BN, H), blk),
            pl.BlockSpec((BN, H), blk),
        ],
        out_shape=[
            jax.ShapeDtypeStruct((N, H), _f32),
            jax.ShapeDtypeStruct((N, H), _f32),
            jax.ShapeDtypeStruct((N, H), _f32),
            jax.ShapeDtypeStruct((N, H), _f32),
        ],
        compiler_params=pltpu.CompilerParams(
            dimension_semantics=("arbitrary",)),
    )(h1, xpad, aggha, agghb, aggxa, aggxb, wh1a, wh1b, bh1, wh2, bh2, a1, b1)


def _node1_body(h2_ref, aggha_ref, agghb_ref, wh1a_ref, wh1b_ref, bh1_ref,
                wh2_ref, bh2_ref, wout_ref, bout_ref, wfin_ref, bfin_ref,
                out_ref):
    h2 = h2_ref[...]
    aggh = ((aggha_ref[0] + aggha_ref[1]) + (agghb_ref[0] + agghb_ref[1]))
    pre = (jnp.dot(h2, wh1a_ref[...], preferred_element_type=_f32) +
           jnp.dot(aggh, wh1b_ref[...], preferred_element_type=_f32) +
           bh1_ref[...])
    upd = jnp.dot(_silu(pre), wh2_ref[...],
                  preferred_element_type=_f32) + bh2_ref[...]
    hf = h2 + upd
    ho = jnp.dot(hf, wout_ref[...], preferred_element_type=_f32) + bout_ref[...]
    out_ref[...] = jnp.dot(ho, wfin_ref[...],
                           preferred_element_type=_f32) + bfin_ref[...]


def _tc_node1(h2, aggha, agghb, wh1a, wh1b, bh1, wh2, bh2, wout, bout,
              wfinr, bfin):
    grid = (N // BN,)
    blk = lambda i: (i, 0)
    blk3c = lambda i: (0, i, 0)
    full = lambda i: (0, 0)
    return pl.pallas_call(
        _node1_body,
        grid=grid,
        in_specs=[
            pl.BlockSpec((BN, H), blk),
            pl.BlockSpec((NC, BN, H), blk3c),
            pl.BlockSpec((NC, BN, H), blk3c),
            pl.BlockSpec((H, H), full),
            pl.BlockSpec((H, H), full),
            pl.BlockSpec((1, H), full),
            pl.BlockSpec((H, H), full),
            pl.BlockSpec((1, H), full),
            pl.BlockSpec((H, H), full),
            pl.BlockSpec((1, H), full),
            pl.BlockSpec((H, 1), full),
            pl.BlockSpec((1, 1), full),
        ],
        out_specs=[pl.BlockSpec((BN, 1), blk)],
        out_shape=[jax.ShapeDtypeStruct((N, 1), _f32)],
        compiler_params=pltpu.CompilerParams(
            dimension_semantics=("arbitrary",)),
    )(h2, aggha, agghb, wh1a, wh1b, bh1, wh2, bh2, wout, bout, wfinr,
      bfin)[0]


def _sc_gather(tp, tq, tx, src2d, dst2d):
    @functools.partial(
        pl.kernel,
        out_type=(jax.ShapeDtypeStruct((EH, H), _f32),
                  jax.ShapeDtypeStruct((EH, H), _f32),
                  jax.ShapeDtypeStruct((EH, H), _f32),
                  jax.ShapeDtypeStruct((EH, H), _f32)),
        mesh=_mesh(),
        scratch_types=[
            pltpu.VMEM((GRPT, 1, GCH), jnp.int32),
            pltpu.VMEM((GRPT, 1, GCH), jnp.int32),
            pltpu.VMEM((GCH, H), _f32),
            pltpu.VMEM((GCH, H), _f32),
            pltpu.VMEM((GCH, H), _f32),
            pltpu.VMEM((GCH, H), _f32),
            pltpu.VMEM((GCH, H), _f32),
            pltpu.VMEM((GCH, H), _f32),
            pltpu.VMEM((GCH, H), _f32),
            pltpu.VMEM((GCH, H), _f32),
            pltpu.SemaphoreType.DMA,
            pltpu.SemaphoreType.DMA,
            pltpu.SemaphoreType.DMA,
            pltpu.SemaphoreType.DMA,
            pltpu.SemaphoreType.DMA,
            pltpu.SemaphoreType.DMA,
            pltpu.SemaphoreType.DMA,
            pltpu.SemaphoreType.DMA,
            pltpu.SemaphoreType.DMA,
            pltpu.SemaphoreType.DMA,
            pltpu.SemaphoreType.DMA,
            pltpu.SemaphoreType.DMA,
            pltpu.SemaphoreType.DMA,
            pltpu.SemaphoreType.DMA,
            pltpu.SemaphoreType.DMA,
            pltpu.SemaphoreType.DMA,
        ],
    )
    def k(tp_hbm, tq_hbm, tx_hbm, src_hbm, dst_hbm,
          gp_hbm, gq_hbm, gxs_hbm, gxd_hbm,
          idxs_v, idxd_v, p0, q0, xs0, xd0, p1, q1, xs1, xd1,
          sp0, sq0, sxs0, sxd0, sp1, sq1, sxs1, sxd1,
          wp0, wq0, wxs0, wxd0, wp1, wq1, wxs1, wxd1):
        cid = jax.lax.axis_index("c")
        sid = jax.lax.axis_index("s")
        wid = sid * NC + cid
        row0 = wid * GRPT
        npairs = jnp.where(wid == NW - 1, GRPT_LAST // 2, GRPT // 2)

        @pl.when(wid < NW - 1)
        def _():
            c1 = pltpu.async_copy(src_hbm.at[pl.ds(row0, GRPT)], idxs_v, sp0)
            c2 = pltpu.async_copy(dst_hbm.at[pl.ds(row0, GRPT)], idxd_v, sq0)
            c1.wait()
            c2.wait()

        @pl.when(wid == NW - 1)
        def _():
            c1 = pltpu.async_copy(src_hbm.at[pl.ds(row0, GRPT_LAST)],
                                  idxs_v.at[pl.ds(0, GRPT_LAST)], sp0)
            c2 = pltpu.async_copy(dst_hbm.at[pl.ds(row0, GRPT_LAST)],
                                  idxd_v.at[pl.ds(0, GRPT_LAST)], sq0)
            c1.wait()
            c2.wait()

        @pl.loop(0, npairs)
        def _(t):
            ca = 2 * t
            cb = 2 * t + 1
            ia = idxs_v.at[ca, 0]
            ja = idxd_v.at[ca, 0]
            ib = idxs_v.at[cb, 0]
            jb = idxd_v.at[cb, 0]
            g1 = pltpu.async_copy(tp_hbm.at[ia], p0, sp0)
            g2 = pltpu.async_copy(tq_hbm.at[ja], q0, sq0)
            g3 = pltpu.async_copy(tx_hbm.at[ia], xs0, sxs0)
            g4 = pltpu.async_copy(tx_hbm.at[ja], xd0, sxd0)
            g5 = pltpu.async_copy(tp_hbm.at[ib], p1, sp1)
            g6 = pltpu.async_copy(tq_hbm.at[jb], q1, sq1)
            g7 = pltpu.async_copy(tx_hbm.at[ib], xs1, sxs1)
            g8 = pltpu.async_copy(tx_hbm.at[jb], xd1, sxd1)
            ba = pl.ds((row0 + ca) * GCH, GCH)
            bb = pl.ds((row0 + cb) * GCH, GCH)
            g1.wait()
            w1 = pltpu.async_copy(p0, gp_hbm.at[ba], wp0)
            g2.wait()
            w2 = pltpu.async_copy(q0, gq_hbm.at[ba], wq0)
            g3.wait()
            w3 = pltpu.async_copy(xs0, gxs_hbm.at[ba], wxs0)
            g4.wait()
            w4 = pltpu.async_copy(xd0, gxd_hbm.at[ba], wxd0)
            g5.wait()
            w5 = pltpu.async_copy(p1, gp_hbm.at[bb], wp1)
            g6.wait()
            w6 = pltpu.async_copy(q1, gq_hbm.at[bb], wq1)
            g7.wait()
            w7 = pltpu.async_copy(xs1, gxs_hbm.at[bb], wxs1)
            g8.wait()
            w8 = pltpu.async_copy(xd1, gxd_hbm.at[bb], wxd1)
            w1.wait()
            w2.wait()
            w3.wait()
            w4.wait()
            w5.wait()
            w6.wait()
            w7.wait()
            w8.wait()

    return k(tp, tq, tx, src2d, dst2d)


def _sc_scatter(m, src2d, zh):
    @functools.partial(
        pl.kernel,
        out_type=jax.ShapeDtypeStruct((NC, N, H), _f32),
        mesh=_mesh(),
        scratch_types=[
            pltpu.VMEM((SRPT, 1, SCH), jnp.int32),
            pltpu.VMEM((SCH, H), _f32),
            pltpu.VMEM((SCH, H), _f32),
            pltpu.VMEM_SHARED((N, H), _f32),
            pltpu.SemaphoreType.DMA,
            pltpu.SemaphoreType.DMA,
            pltpu.SemaphoreType.DMA,
            pltpu.SemaphoreType.DMA,
        ],
    )
    def k(m_hbm, src_hbm, zh_hbm, aggh_hbm, idx_v, mb0, mb1, aggh_s,
          sl0, sl1, ss0, ss1):
        cid = jax.lax.axis_index("c")
        sid = jax.lax.axis_index("s")
        wid = sid * NC + cid
        row0 = wid * SRPT
        npairs = jnp.where(wid == NW - 1, SRPT_LAST // 2, SRPT // 2)

        @pl.when(wid < NW - 1)
        def _():
            pltpu.async_copy(src_hbm.at[pl.ds(row0, SRPT)], idx_v, sl0).wait()

        @pl.when(wid == NW - 1)
        def _():
            pltpu.async_copy(src_hbm.at[pl.ds(row0, SRPT_LAST)],
                             idx_v.at[pl.ds(0, SRPT_LAST)], sl0).wait()

        r0 = sid * NPT
        pltpu.sync_copy(zh_hbm.at[pl.ds(r0, NPT)], aggh_s.at[pl.ds(r0, NPT)])

        @pl.when(sid == 0)
        def _():
            pltpu.sync_copy(zh_hbm.at[pl.ds(NS * NPT, NREM)],
                            aggh_s.at[pl.ds(NS * NPT, NREM)])

        plsc.subcore_barrier()

        @pl.loop(0, npairs)
        def _(t):
            ca = 2 * t
            cb = 2 * t + 1
            la = pltpu.async_copy(
                m_hbm.at[pl.ds((row0 + ca) * SCH, SCH)], mb0, sl0)
            lb = pltpu.async_copy(
                m_hbm.at[pl.ds((row0 + cb) * SCH, SCH)], mb1, sl1)
            la.wait()
            sa = pltpu.async_copy(mb0, aggh_s.at[idx_v.at[ca, 0]], ss0,
                                  add=True)
            lb.wait()
            sb = pltpu.async_copy(mb1, aggh_s.at[idx_v.at[cb, 0]], ss1,
                                  add=True)
            sa.wait()
            sb.wait()

        plsc.subcore_barrier()
        pltpu.sync_copy(aggh_s.at[pl.ds(r0, NPT)],
                        aggh_hbm.at[cid, pl.ds(r0, NPT)])

        @pl.when(sid == 0)
        def _():
            pltpu.sync_copy(aggh_s.at[pl.ds(NS * NPT, NREM)],
                            aggh_hbm.at[cid, pl.ds(NS * NPT, NREM)])

    return k(m, src2d, zh)


def kernel(h, x, edge_index, edge_attr, W_in, b_in, We1, be1, We2, be2,
           Wx1, bx1, Wx2, bx2, Wh1, bh1, Wh2, bh2, W_out, b_out, W_fin, b_fin):
    srcg = edge_index[0].reshape(E // GCH, 1, GCH)
    dstg = edge_index[1].reshape(E // GCH, 1, GCH)
    srcs = edge_index[0].reshape(E // SCH, 1, SCH)
    xpad = jnp.pad(x, ((0, 0), (0, H - x.shape[1])))

    a_l = [We1[l][:H] for l in range(2)]
    b_l = [We1[l][H:2 * H] for l in range(2)]
    wr_l = [We1[l][2 * H:2 * H + 1] for l in range(2)]
    wea_l = [We1[l][2 * H + 1:] for l in range(2)]
    be1_l = [be1[l].reshape(1, H) for l in range(2)]
    be2_l = [be2[l].reshape(1, H) for l in range(2)]
    bx1_0 = bx1[0].reshape(1, H)
    wx2t_0 = jnp.broadcast_to(Wx2[0], (H, H))
    bx2_0 = bx2[0].reshape(1, 1)
    wh1a_l = [Wh1[l][:H] for l in range(2)]
    wh1b_l = [Wh1[l][H:] for l in range(2)]
    bh1_l = [bh1[l].reshape(1, H) for l in range(2)]
    bh2_l = [bh2[l].reshape(1, H) for l in range(2)]
    b_in2 = b_in.reshape(1, H)
    b_out2 = b_out.reshape(1, H)
    bfin2 = b_fin.reshape(1, 1)
    zh = jnp.zeros((N, H), _f32)
    ones_c = jnp.ones((H, H), _f32)

    srcgA, srcgB = srcg[:GROWS_H], srcg[GROWS_H:]
    dstgA, dstgB = dstg[:GROWS_H], dstg[GROWS_H:]
    srcsA, srcsB = srcs[:SROWS_H], srcs[SROWS_H:]
    eaA, eaB = edge_attr[:EH], edge_attr[EH:]

    h1, tp0, tq0 = _tc_prep(h, W_in, b_in2, a_l[0], b_l[0])
    gpA, gqA, gxsA, gxdA = _sc_gather(tp0, tq0, xpad, srcgA, dstgA)
    mA, tA = _tc_edge0(gpA, gxsA, gqA, gxdA, eaA, wr_l[0], wea_l[0],
                       be1_l[0], We2[0], be2_l[0], Wx1[0], bx1_0, wx2t_0,
                       bx2_0, ones_c)
    gpB, gqB, gxsB, gxdB = _sc_gather(tp0, tq0, xpad, srcgB, dstgB)
    mB, tB = _tc_edge0(gpB, gxsB, gqB, gxdB, eaB, wr_l[0], wea_l[0],
                       be1_l[0], We2[0], be2_l[0], Wx1[0], bx1_0, wx2t_0,
                       bx2_0, ones_c)
    agghA = _sc_scatter(mA, srcsA, zh)
    aggxA = _sc_scatter(tA, srcsA, zh)
    agghB = _sc_scatter(mB, srcsB, zh)
    aggxB = _sc_scatter(tB, srcsB, zh)
    h2, tp1, tq1, tx1 = _tc_node0(h1, xpad, agghA, agghB, aggxA, aggxB,
                                  wh1a_l[0], wh1b_l[0], bh1_l[0], Wh2[0],
                                  bh2_l[0], a_l[1], b_l[1])

    gp1A, gq1A, gxs1A, gxd1A = _sc_gather(tp1, tq1, tx1, srcgA, dstgA)
    m1A = _tc_edge1(gp1A, gxs1A, gq1A, gxd1A, eaA, wr_l[1], wea_l[1],
                    be1_l[1], We2[1], be2_l[1], ones_c)
    gp1B, gq1B, gxs1B, gxd1B = _sc_gather(tp1, tq1, tx1, srcgB, dstgB)
    m1B = _tc_edge1(gp1B, gxs1B, gq1B, gxd1B, eaB, wr_l[1], wea_l[1],
                    be1_l[1], We2[1], be2_l[1], ones_c)
    aggh1A = _sc_scatter(m1A, srcsA, zh)
    aggh1B = _sc_scatter(m1B, srcsB, zh)
    out = _tc_node1(h2, aggh1A, aggh1B, wh1a_l[1], wh1b_l[1], bh1_l[1],
                    Wh2[1], bh2_l[1], W_out, b_out2, W_fin, bfin2)
    return out
